# Initial kernel scaffold; baseline (speedup 1.0000x reference)
import jax
import jax.numpy as jnp
from jax.experimental import pallas as pl


def kernel(patches, task_tokens):
    raise NotImplementedError("write your pallas kernel here")


# speedup vs baseline: 3.6496x; 3.6496x over previous
import functools
import math

import jax
import jax.numpy as jnp
from jax.experimental import pallas as pl


def _rms(x, eps=1e-6):
    var = jnp.mean(x * x, axis=-1, keepdims=True)
    return x * jax.lax.rsqrt(var + eps)


def _core(p_ref, t_ref, o_ref):
    p = p_ref[0]
    t = t_ref[0]
    d = p.shape[-1]
    scale = 1.0 / math.sqrt(d)

    pn = _rms(p)
    tn = _rms(t)

    logits = jax.lax.dot_general(
        pn, tn, (((1,), (1,)), ((), ())),
        preferred_element_type=jnp.float32) * scale
    attn = jax.nn.softmax(logits, axis=-1)
    q = jax.lax.dot_general(
        attn, tn, (((1,), (0,)), ((), ())),
        preferred_element_type=jnp.float32)
    qn = _rms(q)
    score = jax.lax.dot_general(
        qn, pn, (((1,), (1,)), ((), ())),
        preferred_element_type=jnp.float32) * scale

    idx = jnp.argmax(score, axis=-1)
    cols = jax.lax.broadcasted_iota(jnp.int32, score.shape, 1)
    o_ref[0] = jnp.where(cols == idx[:, None], 1.0, 0.0).astype(jnp.float32)


@functools.partial(jax.jit, static_argnames=())
def kernel(patches, task_tokens):
    b, n, d = patches.shape
    t = task_tokens.shape[1]
    return pl.pallas_call(
        _core,
        grid=(b,),
        in_specs=[
            pl.BlockSpec((1, n, d), lambda i: (i, 0, 0)),
            pl.BlockSpec((1, t, d), lambda i: (i, 0, 0)),
        ],
        out_specs=pl.BlockSpec((1, n, n), lambda i: (i, 0, 0)),
        out_shape=jax.ShapeDtypeStruct((b, n, n), jnp.float32),
    )(patches, task_tokens)
